# baseline (speedup 1.0000x reference)
import functools

import jax
import jax.numpy as jnp
import numpy as np
from jax.experimental import pallas as pl
from jax.experimental.pallas import tpu as pltpu
from jax.experimental.shard_map import shard_map
from jax.sharding import Mesh, PartitionSpec as P


def _acc_kernel(off_ref, z_ref, y_ref, out_ref):
    x = z_ref[:]
    yb = y_ref[:]
    n = x.shape[0]
    bj = yb.shape[0]
    off = off_ref[0]
    num = jax.lax.dot_general(
        x, yb,
        dimension_numbers=(((1,), (1,)), ((), ())),
        preferred_element_type=jnp.float32,
    )
    xn = jnp.sqrt(jnp.sum(x * x, axis=1))
    yn = jnp.sqrt(jnp.sum(yb * yb, axis=1))
    denom = jnp.maximum(xn[:, None] * yn[None, :], 1e-8)
    simt = num / denom
    row = jax.lax.broadcasted_iota(jnp.int32, (n, bj), 0)
    col = jax.lax.broadcasted_iota(jnp.int32, (n, bj), 1) + off
    d = jnp.sum(jnp.where(row == col, simt, 0.0), axis=0, keepdims=True)
    beats = (simt > d) | ((simt == d) & (row < col))
    cnt = jnp.sum(jnp.where(beats, 1.0, 0.0), axis=0, keepdims=True)
    top1 = jnp.sum(jnp.where(cnt == 0.0, 1.0, 0.0), axis=1, keepdims=True)
    top10 = jnp.sum(jnp.where(cnt < 10.0, 1.0, 0.0), axis=1, keepdims=True)
    out_ref[...] = jnp.concatenate([top1, top10], axis=1) * (1.0 / n)


def _local_accuracies(Z, Yb, off):
    return pl.pallas_call(
        _acc_kernel,
        grid_spec=pltpu.PrefetchScalarGridSpec(num_scalar_prefetch=1),
        out_shape=jax.ShapeDtypeStruct((1, 2), jnp.float32),
    )(off, Z, Yb)


def _sharded_kernel(Z, Y, mesh):
    def per_device(z, yb):
        off = jax.lax.axis_index("d").astype(jnp.int32) * yb.shape[0]
        out = _local_accuracies(z, yb, off[None])
        return jax.lax.psum(out, "d")

    f = shard_map(
        per_device,
        mesh=mesh,
        in_specs=(P(), P("d", None)),
        out_specs=P(),
        check_rep=False,
    )
    out = f(Z, Y)
    return (out[0, 0], out[0, 1])


def kernel(Z, Y):
    devs = jax.devices()
    if len(devs) >= 2 and Y.shape[0] % 2 == 0:
        mesh = Mesh(np.array(devs[:2]), ("d",))
        return _sharded_kernel(Z, Y, mesh)
    out = _local_accuracies(Z, Y, jnp.zeros((1,), jnp.int32))
    return (out[0, 0], out[0, 1])


# speedup vs baseline: 46.1593x; 46.1593x over previous
import jax
import jax.numpy as jnp
from jax.experimental import pallas as pl
from jax.experimental.pallas import tpu as pltpu

_N = 1024
_H = _N // 2


def _half(x, xn, yh, off, n):
    num = jax.lax.dot_general(
        x, yh,
        dimension_numbers=(((1,), (1,)), ((), ())),
        preferred_element_type=jnp.float32,
    )
    yn = jnp.sqrt(jnp.sum(yh * yh, axis=1))
    denom = jnp.maximum(xn * yn[None, :], 1e-8)
    simt = num / denom
    row = jax.lax.broadcasted_iota(jnp.int32, (n, _H), 0)
    col = jax.lax.broadcasted_iota(jnp.int32, (n, _H), 1) + off
    d = jnp.sum(jnp.where(row == col, simt, 0.0), axis=0, keepdims=True)
    beats = (simt > d) | ((simt == d) & (row < col))
    cnt = jnp.sum(jnp.where(beats, 1.0, 0.0), axis=0, keepdims=True)
    top1 = jnp.sum(jnp.where(cnt == 0.0, 1.0, 0.0), axis=1, keepdims=True)
    top10 = jnp.sum(jnp.where(cnt < 10.0, 1.0, 0.0), axis=1, keepdims=True)
    return top1, top10


def _acc_kernel(z_hbm, y_hbm, out_ref, xv, yv, sx, sy0, sy1):
    cx = pltpu.make_async_copy(z_hbm, xv, sx)
    cx.start()
    cy0 = pltpu.make_async_copy(y_hbm.at[pl.ds(0, _H), :], yv.at[pl.ds(0, _H), :], sy0)
    cy0.start()
    cy1 = pltpu.make_async_copy(y_hbm.at[pl.ds(_H, _H), :], yv.at[pl.ds(_H, _H), :], sy1)
    cy1.start()

    cx.wait()
    x = xv[...]
    xn = jnp.sqrt(jnp.sum(x * x, axis=1))[:, None]

    cy0.wait()
    t1a, t10a = _half(x, xn, yv[pl.ds(0, _H), :], 0, _N)
    cy1.wait()
    t1b, t10b = _half(x, xn, yv[pl.ds(_H, _H), :], _H, _N)

    top1 = t1a + t1b
    top10 = t10a + t10b
    out_ref[...] = jnp.concatenate([top1, top10], axis=1) * (1.0 / _N)


def kernel(Z, Y):
    out = pl.pallas_call(
        _acc_kernel,
        in_specs=[
            pl.BlockSpec(memory_space=pltpu.MemorySpace.HBM),
            pl.BlockSpec(memory_space=pltpu.MemorySpace.HBM),
        ],
        out_specs=pl.BlockSpec(memory_space=pltpu.MemorySpace.VMEM),
        out_shape=jax.ShapeDtypeStruct((1, 2), jnp.float32),
        scratch_shapes=[
            pltpu.VMEM((_N, _N), jnp.float32),
            pltpu.VMEM((_N, _N), jnp.float32),
            pltpu.SemaphoreType.DMA,
            pltpu.SemaphoreType.DMA,
            pltpu.SemaphoreType.DMA,
        ],
    )(Z, Y)
    return (out[0, 0], out[0, 1])


# speedup vs baseline: 51.1258x; 1.1076x over previous
import jax
import jax.numpy as jnp
from jax.experimental import pallas as pl
from jax.experimental.pallas import tpu as pltpu

_N = 1024
_H = _N // 2


def _quad(xh, xnh, yh, row_off, col_off):
    num = jax.lax.dot_general(
        xh, yh,
        dimension_numbers=(((1,), (1,)), ((), ())),
        preferred_element_type=jnp.float32,
    )
    yn = jnp.sqrt(jnp.sum(yh * yh, axis=1))
    denom = jnp.maximum(xnh * yn[None, :], 1e-8)
    simt = num / denom
    row = jax.lax.broadcasted_iota(jnp.int32, (_H, _H), 0) + row_off
    col = jax.lax.broadcasted_iota(jnp.int32, (_H, _H), 1) + col_off
    if row_off == col_off:
        d = jnp.sum(jnp.where(row == col, simt, 0.0), axis=0, keepdims=True)
    else:
        d = None
    return simt, row, col, d


def _count(simt, row, col, d):
    beats = (simt > d) | ((simt == d) & (row < col))
    return jnp.sum(jnp.where(beats, 1.0, 0.0), axis=0, keepdims=True)


def _acc_kernel(z_hbm, y_hbm, out_ref, xv, yv, sx0, sx1, sy0, sy1):
    cx0 = pltpu.make_async_copy(z_hbm.at[pl.ds(0, _H), :], xv.at[pl.ds(0, _H), :], sx0)
    cx0.start()
    cy0 = pltpu.make_async_copy(y_hbm.at[pl.ds(0, _H), :], yv.at[pl.ds(0, _H), :], sy0)
    cy0.start()
    cx1 = pltpu.make_async_copy(z_hbm.at[pl.ds(_H, _H), :], xv.at[pl.ds(_H, _H), :], sx1)
    cx1.start()
    cy1 = pltpu.make_async_copy(y_hbm.at[pl.ds(_H, _H), :], yv.at[pl.ds(_H, _H), :], sy1)
    cy1.start()

    cx0.wait()
    x0 = xv[pl.ds(0, _H), :]
    xn0 = jnp.sqrt(jnp.sum(x0 * x0, axis=1))[:, None]

    cy0.wait()
    y0 = yv[pl.ds(0, _H), :]
    s00, r00, c00, d0 = _quad(x0, xn0, y0, 0, 0)
    cnt0 = _count(s00, r00, c00, d0)

    cx1.wait()
    x1 = xv[pl.ds(_H, _H), :]
    xn1 = jnp.sqrt(jnp.sum(x1 * x1, axis=1))[:, None]
    s10, r10, c10, _ = _quad(x1, xn1, y0, _H, 0)
    cnt0 = cnt0 + _count(s10, r10, c10, d0)
    top1 = jnp.sum(jnp.where(cnt0 == 0.0, 1.0, 0.0), axis=1, keepdims=True)
    top10 = jnp.sum(jnp.where(cnt0 < 10.0, 1.0, 0.0), axis=1, keepdims=True)

    cy1.wait()
    y1 = yv[pl.ds(_H, _H), :]
    s11, r11, c11, d1 = _quad(x1, xn1, y1, _H, _H)
    cnt1 = _count(s11, r11, c11, d1)
    s01, r01, c01, _ = _quad(x0, xn0, y1, 0, _H)
    cnt1 = cnt1 + _count(s01, r01, c01, d1)
    top1 = top1 + jnp.sum(jnp.where(cnt1 == 0.0, 1.0, 0.0), axis=1, keepdims=True)
    top10 = top10 + jnp.sum(jnp.where(cnt1 < 10.0, 1.0, 0.0), axis=1, keepdims=True)

    out_ref[...] = jnp.concatenate([top1, top10], axis=1) * (1.0 / _N)


def kernel(Z, Y):
    out = pl.pallas_call(
        _acc_kernel,
        in_specs=[
            pl.BlockSpec(memory_space=pltpu.MemorySpace.HBM),
            pl.BlockSpec(memory_space=pltpu.MemorySpace.HBM),
        ],
        out_specs=pl.BlockSpec(memory_space=pltpu.MemorySpace.VMEM),
        out_shape=jax.ShapeDtypeStruct((1, 2), jnp.float32),
        scratch_shapes=[
            pltpu.VMEM((_N, _N), jnp.float32),
            pltpu.VMEM((_N, _N), jnp.float32),
            pltpu.SemaphoreType.DMA,
            pltpu.SemaphoreType.DMA,
            pltpu.SemaphoreType.DMA,
            pltpu.SemaphoreType.DMA,
        ],
    )(Z, Y)
    return (out[0, 0], out[0, 1])
